# baseline (speedup 1.0000x reference)
import functools

import jax
import jax.numpy as jnp
from jax import lax
from jax.experimental import pallas as pl
from jax.experimental.pallas import tpu as pltpu
from jax.experimental.pallas import tpu_sc as plsc

N = 10000
E = 320000
EPS = 1e-5

NC = 2
NS = 16
NW = NC * NS
CH = 128
NCHUNK = 80
EP = NW * NCHUNK * CH
NPAD = 10112
RPW = NPAD // NS

_MESH = plsc.VectorSubcoreMesh(core_axis_name="c", subcore_axis_name="s")


NBUF = 8
LEAD = 4


@functools.lru_cache(maxsize=None)
def _make_sc_scatter(F):
  @functools.partial(
      pl.kernel,
      out_type=jax.ShapeDtypeStruct((NC, NPAD, F), jnp.float32),
      mesh=_MESH,
      scratch_types=[
          pltpu.VMEM((NCHUNK, CH), jnp.int32),
          pltpu.VMEM((NCHUNK, CH), jnp.int32),
          [pltpu.VMEM((CH, F), jnp.float32)] * NBUF,
          [pltpu.SemaphoreType.DMA] * NBUF,
          [pltpu.SemaphoreType.DMA] * NBUF,
          pltpu.VMEM_SHARED((NPAD, F), jnp.float32),
          pltpu.VMEM_SHARED((NPAD, F if F <= 32 else 8), jnp.float32),
      ],
      compiler_params=pltpu.CompilerParams(use_tc_tiling_on_sc=False),
  )
  def sc_scatter(h_hbm, s_hbm, t_hbm, z_hbm, out_hbm,
                 sv, tv, bufs, gsems, ssems, acc, hsp):
    cid = lax.axis_index("c")
    sid = lax.axis_index("s")
    wid = sid * NC + cid
    pltpu.sync_copy(s_hbm.at[wid], sv)
    pltpu.sync_copy(t_hbm.at[wid], tv)
    pltpu.sync_copy(z_hbm.at[pl.ds(sid * RPW, RPW)],
                    acc.at[pl.ds(sid * RPW, RPW)])
    if F <= 32:
      pltpu.sync_copy(h_hbm.at[pl.ds(sid * RPW, RPW)],
                      hsp.at[pl.ds(sid * RPW, RPW)])
    plsc.subcore_barrier()
    hsrc = hsp if F <= 32 else h_hbm

    def wait_dma(buf, sem):
      pltpu.make_async_copy(z_hbm.at[pl.ds(0, CH)], buf, sem).wait()

    for k in range(LEAD):
      b = k % NBUF
      pltpu.async_copy(hsrc.at[sv.at[k]], bufs[b], gsems[b])

    @pl.loop(0, NCHUNK, step=NBUF)
    def _(j):
      for u in range(NBUF):
        k = j + u
        b = u
        bn = (u + LEAD) % NBUF

        @pl.when(k + LEAD < NCHUNK)
        def _():
          @pl.when(k + LEAD >= NBUF)
          def _():
            wait_dma(bufs[bn], ssems[bn])
          pltpu.async_copy(hsrc.at[sv.at[k + LEAD]], bufs[bn], gsems[bn])

        wait_dma(bufs[b], gsems[b])
        pltpu.async_copy(bufs[b], acc.at[tv.at[k]], ssems[b], add=True)

    for b in range(NBUF):
      wait_dma(bufs[b], ssems[b])

    plsc.subcore_barrier()
    pltpu.sync_copy(acc.at[pl.ds(sid * RPW, RPW)],
                    out_hbm.at[cid].at[pl.ds(sid * RPW, RPW)])

  return sc_scatter


NCHUNK2 = 2 * NCHUNK


@functools.lru_cache(maxsize=None)
def _make_sc_scatter_fsplit():
  @functools.partial(
      pl.kernel,
      out_type=jax.ShapeDtypeStruct((NC, NPAD, 32), jnp.float32),
      mesh=_MESH,
      scratch_types=[
          pltpu.VMEM((NCHUNK2, CH), jnp.int32),
          pltpu.VMEM((NCHUNK2, CH), jnp.int32),
          [pltpu.VMEM((CH, 32), jnp.float32)] * NBUF,
          [pltpu.SemaphoreType.DMA] * NBUF,
          [pltpu.SemaphoreType.DMA] * NBUF,
          pltpu.VMEM_SHARED((NPAD, 32), jnp.float32),
          pltpu.VMEM_SHARED((NPAD, 32), jnp.float32),
      ],
      compiler_params=pltpu.CompilerParams(use_tc_tiling_on_sc=False),
  )
  def sc_scatter_fs(hlo_hbm, hhi_hbm, s_hbm, t_hbm, z_hbm, out_hbm,
                    sv, tv, bufs, gsems, ssems, acc, hsp):
    cid = lax.axis_index("c")
    sid = lax.axis_index("s")
    pltpu.sync_copy(s_hbm.at[sid], sv)
    pltpu.sync_copy(t_hbm.at[sid], tv)
    pltpu.sync_copy(z_hbm.at[pl.ds(sid * RPW, RPW)],
                    acc.at[pl.ds(sid * RPW, RPW)])

    @pl.when(cid == 0)
    def _():
      pltpu.sync_copy(hlo_hbm.at[pl.ds(sid * RPW, RPW)],
                      hsp.at[pl.ds(sid * RPW, RPW)])

    @pl.when(cid == 1)
    def _():
      pltpu.sync_copy(hhi_hbm.at[pl.ds(sid * RPW, RPW)],
                      hsp.at[pl.ds(sid * RPW, RPW)])

    plsc.subcore_barrier()

    def wait_dma(buf, sem):
      pltpu.make_async_copy(z_hbm.at[pl.ds(0, CH)], buf, sem).wait()

    for k in range(LEAD):
      b = k % NBUF
      pltpu.async_copy(hsp.at[sv.at[k]], bufs[b], gsems[b])

    @pl.loop(0, NCHUNK2, step=NBUF)
    def _(j):
      for u in range(NBUF):
        k = j + u
        b = u
        bn = (u + LEAD) % NBUF

        @pl.when(k + LEAD < NCHUNK2)
        def _():
          @pl.when(k + LEAD >= NBUF)
          def _():
            wait_dma(bufs[bn], ssems[bn])
          pltpu.async_copy(hsp.at[sv.at[k + LEAD]], bufs[bn], gsems[bn])

        wait_dma(bufs[b], gsems[b])
        pltpu.async_copy(bufs[b], acc.at[tv.at[k]], ssems[b], add=True)

    for b in range(NBUF):
      wait_dma(bufs[b], ssems[b])

    plsc.subcore_barrier()
    pltpu.sync_copy(acc.at[pl.ds(sid * RPW, RPW)],
                    out_hbm.at[cid].at[pl.ds(sid * RPW, RPW)])

  return sc_scatter_fs


@functools.lru_cache(maxsize=None)
def _make_sc_deg():
  @functools.partial(
      pl.kernel,
      out_type=jax.ShapeDtypeStruct((NC, NPAD, 16), jnp.float32),
      mesh=_MESH,
      scratch_types=[
          pltpu.VMEM((NCHUNK, CH), jnp.int32),
          pltpu.VMEM((CH, 16), jnp.float32),
          pltpu.SemaphoreType.DMA,
          pltpu.VMEM_SHARED((NPAD, 16), jnp.float32),
      ],
      compiler_params=pltpu.CompilerParams(use_tc_tiling_on_sc=False),
  )
  def sc_deg(t_hbm, ones_hbm, z_hbm, out_hbm, tv, ones_v, sem, acc):
    cid = lax.axis_index("c")
    sid = lax.axis_index("s")
    wid = sid * NC + cid
    pltpu.sync_copy(t_hbm.at[wid], tv)
    pltpu.sync_copy(ones_hbm, ones_v)
    pltpu.sync_copy(z_hbm.at[pl.ds(sid * RPW, RPW)],
                    acc.at[pl.ds(sid * RPW, RPW)])
    plsc.subcore_barrier()

    @pl.loop(0, NCHUNK)
    def _(j):
      pltpu.async_copy(ones_v, acc.at[tv.at[j]], sem, add=True)

    @pl.loop(0, NCHUNK)
    def _(j):
      pltpu.make_async_copy(z_hbm.at[pl.ds(0, CH)], ones_v, sem).wait()

    plsc.subcore_barrier()
    pltpu.sync_copy(acc.at[pl.ds(sid * RPW, RPW)],
                    out_hbm.at[cid].at[pl.ds(sid * RPW, RPW)])

  return sc_deg


_RB = 1000
_KB = 1920
_NRB = N // _RB
_NKB = 9600 // _KB


def _ffn_body(x_ref, w1_ref, b1_ref, w2_ref, b2_ref, o_ref, acc_ref):
  k = pl.program_id(1)
  h = jnp.maximum(
      jnp.dot(x_ref[...], w1_ref[...], preferred_element_type=jnp.float32)
      + b1_ref[...], 0.0)
  p = jnp.dot(h.astype(jnp.bfloat16), w2_ref[...],
              preferred_element_type=jnp.float32)

  @pl.when(k == 0)
  def _():
    acc_ref[...] = p

  @pl.when(k > 0)
  def _():
    acc_ref[...] += p

  @pl.when(k == _NKB - 1)
  def _():
    o_ref[...] = acc_ref[...] + b2_ref[...]


def _tc_ffn(x, w1f, b1f, w2f, b2f):
  return pl.pallas_call(
      _ffn_body,
      grid=(_NRB, _NKB),
      in_specs=[
          pl.BlockSpec((_RB, 40), lambda i, k: (i, 0)),
          pl.BlockSpec((40, _KB), lambda i, k: (0, k)),
          pl.BlockSpec((1, _KB), lambda i, k: (0, k)),
          pl.BlockSpec((_KB, 40), lambda i, k: (k, 0)),
          pl.BlockSpec((1, 40), lambda i, k: (0, 0)),
      ],
      out_specs=pl.BlockSpec((_RB, 40), lambda i, k: (i, 0)),
      out_shape=jax.ShapeDtypeStruct((N, 40), jnp.float32),
      scratch_shapes=[pltpu.VMEM((_RB, 40), jnp.float32)],
      compiler_params=pltpu.CompilerParams(
          dimension_semantics=("arbitrary", "arbitrary")),
  )(x.astype(jnp.bfloat16), w1f.astype(jnp.bfloat16), b1f,
    w2f.astype(jnp.bfloat16), b2f)


def _rowmask(v):
  rows = lax.broadcasted_iota(jnp.int32, v.shape, 0)
  return jnp.where(rows < N, v, 0.0)


def _first_body(h0_ref, d0_ref, d1_ref, w_ref, hp_ref, dinv_ref):
  deg = d0_ref[:, 0:1] + d1_ref[:, 0:1] + 1.0
  dinv = lax.rsqrt(deg)
  dinv_ref[...] = dinv
  g = jnp.dot(h0_ref[...], w_ref[...], preferred_element_type=jnp.float32)
  gp = jnp.concatenate([g, jnp.zeros((NPAD - N, g.shape[1]), jnp.float32)], 0)
  hp_ref[...] = _rowmask(gp * dinv)


def _tc_first(h0, deg0, deg1, wc1):
  return pl.pallas_call(
      _first_body,
      out_shape=(
          jax.ShapeDtypeStruct((NPAD, 64), jnp.float32),
          jax.ShapeDtypeStruct((NPAD, 1), jnp.float32),
      ),
  )(h0, deg0, deg1, wc1)


@functools.lru_cache(maxsize=None)
def _make_tc_mid(fin_s, fin, fout, fout_s, cat=False):
  def mid_body(a0, a1, hp, dv, b, w, o_ref):
    if cat:
      x = jnp.concatenate([a0[...], a1[...]], 1) + hp[...]
    else:
      x = (a0[...] + a1[...] + hp[...])[:, :fin]
    dinv = dv[...]
    o = jnp.maximum(x * dinv + b[...], 0.0)
    g = jnp.dot(o, w[...], preferred_element_type=jnp.float32) * dinv
    if fout_s > fout:
      g = jnp.concatenate(
          [g, jnp.zeros((NPAD, fout_s - fout), jnp.float32)], 1)
    o_ref[...] = _rowmask(g)

  return pl.pallas_call(
      mid_body,
      out_shape=jax.ShapeDtypeStruct((NPAD, fout_s), jnp.float32),
  )


def _last_body(a0, a1, hp, dv, b, o_ref):
  x = (a0[...] + a1[...] + hp[...])[:N, :4]
  o_ref[...] = jnp.maximum(x * dv[:N, :] + b[...], 0.0)


def _tc_last(acc0, acc1, hp, dinv, b):
  return pl.pallas_call(
      _last_body,
      out_shape=jax.ShapeDtypeStruct((N, 4), jnp.float32),
  )(acc0, acc1, hp, dinv, b)


def _fc_body(a_ref, w_ref, b_ref, o_ref):
  o_ref[...] = (
      jnp.dot(a_ref[...], w_ref[...], preferred_element_type=jnp.float32)
      + b_ref[...])


def _tc_fc(a, w, b):
  return pl.pallas_call(
      _fc_body,
      out_shape=jax.ShapeDtypeStruct((a.shape[0], w.shape[1]), jnp.float32),
  )(a, w, b)


def kernel(x, edge_index, batch, W1, b1, g1, be1, W2, b2, g2, be2,
           Wc1, bc1, Wc2, bc2, Wc3, bc3, Wc4, bc4, Wc5, bc5, Wfc, bfc):
  f32 = jnp.float32
  c = 1.0 / jnp.sqrt(jnp.asarray(1.0 + EPS, f32))
  w1f = W1 * (c * g1)[None, :]
  b1f = (b1 * c * g1 + be1)[None, :]
  w2f = W2 * (c * g2)[None, :]
  b2f = (b2 * c * g2 + be2)[None, :]

  s = edge_index[0].astype(jnp.int32)
  t = edge_index[1].astype(jnp.int32)
  pad = jnp.full((EP - E,), N, jnp.int32)
  s_r = jnp.concatenate([s, pad]).reshape(NW, NCHUNK, CH)
  t_r = jnp.concatenate([t, pad]).reshape(NW, NCHUNK, CH)
  s_r2 = s_r.reshape(NS, NCHUNK2, CH)
  t_r2 = t_r.reshape(NS, NCHUNK2, CH)

  ones16 = jnp.ones((CH, 16), f32)
  z16 = jnp.zeros((NPAD, 16), f32)

  degp = _make_sc_deg()(t_r, ones16, z16)
  h0 = _tc_ffn(x, w1f, b1f, w2f, b2f)
  hp, dinv = _tc_first(h0, degp[0], degp[1], Wc1)

  z32 = jnp.zeros((NPAD, 32), f32)
  accp = _make_sc_scatter_fsplit()(hp[:, :32], hp[:, 32:], s_r2, t_r2, z32)
  hp = _make_tc_mid(64, 64, 32, 32, cat=True)(
      accp[0], accp[1], hp, dinv, bc1[None, :], Wc2)

  dims = [(32, 32, 16, 16, bc2, Wc3),
          (16, 16, 8, 16, bc3, Wc4),
          (16, 8, 4, 16, bc4, Wc5)]
  for fin_s, fin, fout, fout_s, bcl, wcl in dims:
    accp = _make_sc_scatter(fin_s)(
        hp, s_r, t_r, jnp.zeros((NPAD, fin_s), f32))
    hp = _make_tc_mid(fin_s, fin, fout, fout_s)(
        accp[0], accp[1], hp, dinv, bcl[None, :], wcl)

  accp = _make_sc_scatter(16)(hp, s_r, t_r, z16)
  o5 = _tc_last(accp[0], accp[1], hp, dinv, bc5[None, :])

  o5r = o5.reshape(-1, 1000 * 4)
  return _tc_fc(o5r, Wfc, bfc[None, :])


# speedup vs baseline: 1.0998x; 1.0998x over previous
import functools

import jax
import jax.numpy as jnp
from jax import lax
from jax.experimental import pallas as pl
from jax.experimental.pallas import tpu as pltpu
from jax.experimental.pallas import tpu_sc as plsc

N = 10000
E = 320000
EPS = 1e-5

NC = 2
NS = 16
NW = NC * NS
CH = 128
NCHUNK = 80
EP = NW * NCHUNK * CH
NPAD = 10112
RPW = NPAD // NS

_MESH = plsc.VectorSubcoreMesh(core_axis_name="c", subcore_axis_name="s")


NBUF = 8
LEAD = 4


@functools.lru_cache(maxsize=None)
def _make_sc_scatter(F):
  @functools.partial(
      pl.kernel,
      out_type=jax.ShapeDtypeStruct((NC, NPAD, F), jnp.float32),
      mesh=_MESH,
      scratch_types=[
          pltpu.VMEM((NCHUNK, CH), jnp.int32),
          pltpu.VMEM((NCHUNK, CH), jnp.int32),
          [pltpu.VMEM((CH, F), jnp.float32)] * NBUF,
          [pltpu.SemaphoreType.DMA] * NBUF,
          [pltpu.SemaphoreType.DMA] * NBUF,
          pltpu.VMEM_SHARED((NPAD, F), jnp.float32),
          pltpu.VMEM_SHARED((NPAD, F if F <= 32 else 8), jnp.float32),
      ],
      compiler_params=pltpu.CompilerParams(use_tc_tiling_on_sc=False),
  )
  def sc_scatter(h_hbm, s_hbm, t_hbm, z_hbm, out_hbm,
                 sv, tv, bufs, gsems, ssems, acc, hsp):
    cid = lax.axis_index("c")
    sid = lax.axis_index("s")
    wid = sid * NC + cid
    pltpu.sync_copy(s_hbm.at[wid], sv)
    pltpu.sync_copy(t_hbm.at[wid], tv)
    pltpu.sync_copy(z_hbm.at[pl.ds(sid * RPW, RPW)],
                    acc.at[pl.ds(sid * RPW, RPW)])
    if F <= 32:
      pltpu.sync_copy(h_hbm.at[pl.ds(sid * RPW, RPW)],
                      hsp.at[pl.ds(sid * RPW, RPW)])
    plsc.subcore_barrier()
    hsrc = hsp if F <= 32 else h_hbm

    def wait_dma(buf, sem):
      pltpu.make_async_copy(z_hbm.at[pl.ds(0, CH)], buf, sem).wait()

    for k in range(LEAD):
      b = k % NBUF
      pltpu.async_copy(hsrc.at[sv.at[k]], bufs[b], gsems[b])

    @pl.loop(0, NCHUNK, step=NBUF)
    def _(j):
      for u in range(NBUF):
        k = j + u
        b = u
        bn = (u + LEAD) % NBUF

        @pl.when(k + LEAD < NCHUNK)
        def _():
          @pl.when(k + LEAD >= NBUF)
          def _():
            wait_dma(bufs[bn], ssems[bn])
          pltpu.async_copy(hsrc.at[sv.at[k + LEAD]], bufs[bn], gsems[bn])

        wait_dma(bufs[b], gsems[b])
        pltpu.async_copy(bufs[b], acc.at[tv.at[k]], ssems[b], add=True)

    for b in range(NBUF):
      wait_dma(bufs[b], ssems[b])

    plsc.subcore_barrier()
    pltpu.sync_copy(acc.at[pl.ds(sid * RPW, RPW)],
                    out_hbm.at[cid].at[pl.ds(sid * RPW, RPW)])

  return sc_scatter


NCHUNK2 = 2 * NCHUNK


@functools.lru_cache(maxsize=None)
def _make_sc_scatter_fsplit():
  @functools.partial(
      pl.kernel,
      out_type=jax.ShapeDtypeStruct((NC, NPAD, 32), jnp.float32),
      mesh=_MESH,
      scratch_types=[
          pltpu.VMEM((NCHUNK2, CH), jnp.int32),
          pltpu.VMEM((NCHUNK2, CH), jnp.int32),
          [pltpu.VMEM((CH, 32), jnp.float32)] * NBUF,
          [pltpu.SemaphoreType.DMA] * NBUF,
          [pltpu.SemaphoreType.DMA] * NBUF,
          pltpu.VMEM_SHARED((NPAD, 32), jnp.float32),
          pltpu.VMEM_SHARED((NPAD, 32), jnp.float32),
      ],
      compiler_params=pltpu.CompilerParams(use_tc_tiling_on_sc=False),
  )
  def sc_scatter_fs(h_hbm, s_hbm, t_hbm, z_hbm, out_hbm,
                    sv, tv, bufs, gsems, ssems, acc, hsp):
    cid = lax.axis_index("c")
    sid = lax.axis_index("s")
    pltpu.sync_copy(s_hbm.at[sid], sv)
    pltpu.sync_copy(t_hbm.at[sid], tv)
    pltpu.sync_copy(z_hbm.at[pl.ds(sid * RPW, RPW)],
                    acc.at[pl.ds(sid * RPW, RPW)])
    pltpu.sync_copy(h_hbm.at[pl.ds(sid * RPW, RPW), pl.ds(cid * 32, 32)],
                    hsp.at[pl.ds(sid * RPW, RPW)])
    plsc.subcore_barrier()

    def wait_dma(buf, sem):
      pltpu.make_async_copy(z_hbm.at[pl.ds(0, CH)], buf, sem).wait()

    for k in range(LEAD):
      b = k % NBUF
      pltpu.async_copy(hsp.at[sv.at[k]], bufs[b], gsems[b])

    @pl.loop(0, NCHUNK2, step=NBUF)
    def _(j):
      for u in range(NBUF):
        k = j + u
        b = u
        bn = (u + LEAD) % NBUF

        @pl.when(k + LEAD < NCHUNK2)
        def _():
          @pl.when(k + LEAD >= NBUF)
          def _():
            wait_dma(bufs[bn], ssems[bn])
          pltpu.async_copy(hsp.at[sv.at[k + LEAD]], bufs[bn], gsems[bn])

        wait_dma(bufs[b], gsems[b])
        pltpu.async_copy(bufs[b], acc.at[tv.at[k]], ssems[b], add=True)

    for b in range(NBUF):
      wait_dma(bufs[b], ssems[b])

    plsc.subcore_barrier()
    pltpu.sync_copy(acc.at[pl.ds(sid * RPW, RPW)],
                    out_hbm.at[cid].at[pl.ds(sid * RPW, RPW)])

  return sc_scatter_fs


@functools.lru_cache(maxsize=None)
def _make_sc_deg():
  @functools.partial(
      pl.kernel,
      out_type=jax.ShapeDtypeStruct((NC, NPAD, 16), jnp.float32),
      mesh=_MESH,
      scratch_types=[
          pltpu.VMEM((NCHUNK, CH), jnp.int32),
          pltpu.VMEM((CH, 16), jnp.float32),
          pltpu.SemaphoreType.DMA,
          pltpu.VMEM_SHARED((NPAD, 16), jnp.float32),
      ],
      compiler_params=pltpu.CompilerParams(use_tc_tiling_on_sc=False),
  )
  def sc_deg(t_hbm, ones_hbm, z_hbm, out_hbm, tv, ones_v, sem, acc):
    cid = lax.axis_index("c")
    sid = lax.axis_index("s")
    wid = sid * NC + cid
    pltpu.sync_copy(t_hbm.at[wid], tv)
    pltpu.sync_copy(ones_hbm, ones_v)
    pltpu.sync_copy(z_hbm.at[pl.ds(sid * RPW, RPW)],
                    acc.at[pl.ds(sid * RPW, RPW)])
    plsc.subcore_barrier()

    @pl.loop(0, NCHUNK)
    def _(j):
      pltpu.async_copy(ones_v, acc.at[tv.at[j]], sem, add=True)

    @pl.loop(0, NCHUNK)
    def _(j):
      pltpu.make_async_copy(z_hbm.at[pl.ds(0, CH)], ones_v, sem).wait()

    plsc.subcore_barrier()
    pltpu.sync_copy(acc.at[pl.ds(sid * RPW, RPW)],
                    out_hbm.at[cid].at[pl.ds(sid * RPW, RPW)])

  return sc_deg


_RB = 1000
_KB = 1920
_NRB = N // _RB
_NKB = 9600 // _KB


def _ffn_body(x_ref, w1_ref, b1_ref, w2_ref, b2_ref, o_ref, acc_ref):
  k = pl.program_id(1)
  h = jnp.maximum(
      jnp.dot(x_ref[...], w1_ref[...], preferred_element_type=jnp.float32)
      + b1_ref[...], 0.0)
  p = jnp.dot(h.astype(jnp.bfloat16), w2_ref[...],
              preferred_element_type=jnp.float32)

  @pl.when(k == 0)
  def _():
    acc_ref[...] = p

  @pl.when(k > 0)
  def _():
    acc_ref[...] += p

  @pl.when(k == _NKB - 1)
  def _():
    o_ref[...] = acc_ref[...] + b2_ref[...]


def _tc_ffn(x, w1f, b1f, w2f, b2f):
  return pl.pallas_call(
      _ffn_body,
      grid=(_NRB, _NKB),
      in_specs=[
          pl.BlockSpec((_RB, 40), lambda i, k: (i, 0)),
          pl.BlockSpec((40, _KB), lambda i, k: (0, k)),
          pl.BlockSpec((1, _KB), lambda i, k: (0, k)),
          pl.BlockSpec((_KB, 40), lambda i, k: (k, 0)),
          pl.BlockSpec((1, 40), lambda i, k: (0, 0)),
      ],
      out_specs=pl.BlockSpec((_RB, 40), lambda i, k: (i, 0)),
      out_shape=jax.ShapeDtypeStruct((N, 40), jnp.float32),
      scratch_shapes=[pltpu.VMEM((_RB, 40), jnp.float32)],
      compiler_params=pltpu.CompilerParams(
          dimension_semantics=("parallel", "arbitrary")),
  )(x.astype(jnp.bfloat16), w1f.astype(jnp.bfloat16), b1f,
    w2f.astype(jnp.bfloat16), b2f)


def _rowmask(v):
  rows = lax.broadcasted_iota(jnp.int32, v.shape, 0)
  return jnp.where(rows < N, v, 0.0)


def _first_body(h0_ref, dp_ref, w_ref, hp_ref, dinv_ref):
  deg = dp_ref[0, :, 0:1] + dp_ref[1, :, 0:1] + 1.0
  dinv = lax.rsqrt(deg)
  dinv_ref[...] = dinv
  g = jnp.dot(h0_ref[...], w_ref[...], preferred_element_type=jnp.float32)
  gp = jnp.concatenate([g, jnp.zeros((NPAD - N, g.shape[1]), jnp.float32)], 0)
  hp_ref[...] = _rowmask(gp * dinv)


def _tc_first(h0, degp, wc1):
  return pl.pallas_call(
      _first_body,
      out_shape=(
          jax.ShapeDtypeStruct((NPAD, 64), jnp.float32),
          jax.ShapeDtypeStruct((NPAD, 1), jnp.float32),
      ),
  )(h0, degp, wc1)


@functools.lru_cache(maxsize=None)
def _make_tc_mid(fin_s, fin, fout, fout_s, cat=False):
  def mid_body(ap, hp, dv, b, w, o_ref):
    if cat:
      x = jnp.concatenate([ap[0], ap[1]], 1) + hp[...]
    else:
      x = (ap[0] + ap[1] + hp[...])[:, :fin]
    dinv = dv[...]
    o = jnp.maximum(x * dinv + b[...], 0.0)
    g = jnp.dot(o, w[...], preferred_element_type=jnp.float32) * dinv
    if fout_s > fout:
      g = jnp.concatenate(
          [g, jnp.zeros((NPAD, fout_s - fout), jnp.float32)], 1)
    o_ref[...] = _rowmask(g)

  return pl.pallas_call(
      mid_body,
      out_shape=jax.ShapeDtypeStruct((NPAD, fout_s), jnp.float32),
  )


def _last_body(ap, hp, dv, b, o_ref):
  x = (ap[0] + ap[1] + hp[...])[:N, :4]
  o_ref[...] = jnp.maximum(x * dv[:N, :] + b[...], 0.0)


def _tc_last(accp, hp, dinv, b):
  return pl.pallas_call(
      _last_body,
      out_shape=jax.ShapeDtypeStruct((N, 4), jnp.float32),
  )(accp, hp, dinv, b)


def _fc_body(a_ref, w_ref, b_ref, o_ref):
  o_ref[...] = (
      jnp.dot(a_ref[...], w_ref[...], preferred_element_type=jnp.float32)
      + b_ref[...])


def _tc_fc(a, w, b):
  return pl.pallas_call(
      _fc_body,
      out_shape=jax.ShapeDtypeStruct((a.shape[0], w.shape[1]), jnp.float32),
  )(a, w, b)


def kernel(x, edge_index, batch, W1, b1, g1, be1, W2, b2, g2, be2,
           Wc1, bc1, Wc2, bc2, Wc3, bc3, Wc4, bc4, Wc5, bc5, Wfc, bfc):
  f32 = jnp.float32
  c = 1.0 / jnp.sqrt(jnp.asarray(1.0 + EPS, f32))
  w1f = W1 * (c * g1)[None, :]
  b1f = (b1 * c * g1 + be1)[None, :]
  w2f = W2 * (c * g2)[None, :]
  b2f = (b2 * c * g2 + be2)[None, :]

  s = edge_index[0].astype(jnp.int32)
  t = edge_index[1].astype(jnp.int32)
  pad = jnp.full((EP - E,), N, jnp.int32)
  s_r = jnp.concatenate([s, pad]).reshape(NW, NCHUNK, CH)
  t_r = jnp.concatenate([t, pad]).reshape(NW, NCHUNK, CH)
  s_r2 = s_r.reshape(NS, NCHUNK2, CH)
  t_r2 = t_r.reshape(NS, NCHUNK2, CH)

  ones16 = jnp.ones((CH, 16), f32)
  z16 = jnp.zeros((NPAD, 16), f32)

  degp = _make_sc_deg()(t_r, ones16, z16)
  h0 = _tc_ffn(x, w1f, b1f, w2f, b2f)
  hp, dinv = _tc_first(h0, degp, Wc1)

  z32 = jnp.zeros((NPAD, 32), f32)
  accp = _make_sc_scatter_fsplit()(hp, s_r2, t_r2, z32)
  hp = _make_tc_mid(64, 64, 32, 32, cat=True)(
      accp, hp, dinv, bc1[None, :], Wc2)

  dims = [(32, 32, 16, 16, bc2, Wc3),
          (16, 16, 8, 16, bc3, Wc4),
          (16, 8, 4, 16, bc4, Wc5)]
  for fin_s, fin, fout, fout_s, bcl, wcl in dims:
    accp = _make_sc_scatter(fin_s)(
        hp, s_r, t_r, jnp.zeros((NPAD, fin_s), f32))
    hp = _make_tc_mid(fin_s, fin, fout, fout_s)(
        accp, hp, dinv, bcl[None, :], wcl)

  accp = _make_sc_scatter(16)(hp, s_r, t_r, z16)
  o5 = _tc_last(accp, hp, dinv, bc5[None, :])
  return _tc_fc(o5.reshape(-1, 1000 * 4), Wfc, bfc[None, :])


# speedup vs baseline: 1.1018x; 1.0018x over previous
import functools

import jax
import jax.numpy as jnp
from jax import lax
from jax.experimental import pallas as pl
from jax.experimental.pallas import tpu as pltpu
from jax.experimental.pallas import tpu_sc as plsc

N = 10000
E = 320000
EPS = 1e-5

NC = 2
NS = 16
NW = NC * NS
CH = 128
NCHUNK = 80
EP = NW * NCHUNK * CH
NPAD = 10112
RPW = NPAD // NS

_MESH = plsc.VectorSubcoreMesh(core_axis_name="c", subcore_axis_name="s")


NBUF = 8
LEAD = 4


@functools.lru_cache(maxsize=None)
def _make_sc_scatter(F):
  @functools.partial(
      pl.kernel,
      out_type=jax.ShapeDtypeStruct((NC, NPAD, F), jnp.float32),
      mesh=_MESH,
      scratch_types=[
          pltpu.VMEM((NCHUNK, CH), jnp.int32),
          pltpu.VMEM((NCHUNK, CH), jnp.int32),
          [pltpu.VMEM((CH, F), jnp.float32)] * NBUF,
          [pltpu.SemaphoreType.DMA] * NBUF,
          [pltpu.SemaphoreType.DMA] * NBUF,
          pltpu.VMEM_SHARED((NPAD, F), jnp.float32),
          pltpu.VMEM_SHARED((NPAD, F if F <= 32 else 8), jnp.float32),
      ],
      compiler_params=pltpu.CompilerParams(use_tc_tiling_on_sc=False),
  )
  def sc_scatter(h_hbm, s_hbm, t_hbm, z_hbm, out_hbm,
                 sv, tv, bufs, gsems, ssems, acc, hsp):
    cid = lax.axis_index("c")
    sid = lax.axis_index("s")
    wid = sid * NC + cid
    pltpu.sync_copy(s_hbm.at[wid], sv)
    pltpu.sync_copy(t_hbm.at[wid], tv)
    pltpu.sync_copy(z_hbm.at[pl.ds(sid * RPW, RPW)],
                    acc.at[pl.ds(sid * RPW, RPW)])
    if F <= 32:
      pltpu.sync_copy(h_hbm.at[pl.ds(sid * RPW, RPW)],
                      hsp.at[pl.ds(sid * RPW, RPW)])
    plsc.subcore_barrier()
    hsrc = hsp if F <= 32 else h_hbm

    def wait_dma(buf, sem):
      pltpu.make_async_copy(z_hbm.at[pl.ds(0, CH)], buf, sem).wait()

    for k in range(LEAD):
      b = k % NBUF
      pltpu.async_copy(hsrc.at[sv.at[k]], bufs[b], gsems[b])

    @pl.loop(0, NCHUNK, step=NBUF)
    def _(j):
      for u in range(NBUF):
        k = j + u
        b = u
        bn = (u + LEAD) % NBUF

        @pl.when(k + LEAD < NCHUNK)
        def _():
          @pl.when(k + LEAD >= NBUF)
          def _():
            wait_dma(bufs[bn], ssems[bn])
          pltpu.async_copy(hsrc.at[sv.at[k + LEAD]], bufs[bn], gsems[bn])

        wait_dma(bufs[b], gsems[b])
        pltpu.async_copy(bufs[b], acc.at[tv.at[k]], ssems[b], add=True)

    for b in range(NBUF):
      wait_dma(bufs[b], ssems[b])

    plsc.subcore_barrier()
    pltpu.sync_copy(acc.at[pl.ds(sid * RPW, RPW)],
                    out_hbm.at[cid].at[pl.ds(sid * RPW, RPW)])

  return sc_scatter


NCHUNK2 = 2 * NCHUNK


@functools.lru_cache(maxsize=None)
def _make_sc_scatter_fsplit():
  @functools.partial(
      pl.kernel,
      out_type=jax.ShapeDtypeStruct((NC, NPAD, 32), jnp.float32),
      mesh=_MESH,
      scratch_types=[
          pltpu.VMEM((NCHUNK2, CH), jnp.int32),
          pltpu.VMEM((NCHUNK2, CH), jnp.int32),
          [pltpu.VMEM((CH, 32), jnp.float32)] * NBUF,
          [pltpu.SemaphoreType.DMA] * NBUF,
          [pltpu.SemaphoreType.DMA] * NBUF,
          pltpu.VMEM_SHARED((NPAD, 32), jnp.float32),
          pltpu.VMEM_SHARED((NPAD, 32), jnp.float32),
      ],
      compiler_params=pltpu.CompilerParams(use_tc_tiling_on_sc=False),
  )
  def sc_scatter_fs(h_hbm, s_hbm, t_hbm, z_hbm, out_hbm,
                    sv, tv, bufs, gsems, ssems, acc, hsp):
    cid = lax.axis_index("c")
    sid = lax.axis_index("s")
    pltpu.sync_copy(s_hbm.at[sid], sv)
    pltpu.sync_copy(t_hbm.at[sid], tv)
    pltpu.sync_copy(z_hbm.at[pl.ds(sid * RPW, RPW)],
                    acc.at[pl.ds(sid * RPW, RPW)])
    pltpu.sync_copy(h_hbm.at[pl.ds(sid * RPW, RPW), pl.ds(cid * 32, 32)],
                    hsp.at[pl.ds(sid * RPW, RPW)])
    plsc.subcore_barrier()

    def wait_dma(buf, sem):
      pltpu.make_async_copy(z_hbm.at[pl.ds(0, CH)], buf, sem).wait()

    for k in range(LEAD):
      b = k % NBUF
      pltpu.async_copy(hsp.at[sv.at[k]], bufs[b], gsems[b])

    @pl.loop(0, NCHUNK2, step=NBUF)
    def _(j):
      for u in range(NBUF):
        k = j + u
        b = u
        bn = (u + LEAD) % NBUF

        @pl.when(k + LEAD < NCHUNK2)
        def _():
          @pl.when(k + LEAD >= NBUF)
          def _():
            wait_dma(bufs[bn], ssems[bn])
          pltpu.async_copy(hsp.at[sv.at[k + LEAD]], bufs[bn], gsems[bn])

        wait_dma(bufs[b], gsems[b])
        pltpu.async_copy(bufs[b], acc.at[tv.at[k]], ssems[b], add=True)

    for b in range(NBUF):
      wait_dma(bufs[b], ssems[b])

    plsc.subcore_barrier()
    pltpu.sync_copy(acc.at[pl.ds(sid * RPW, RPW)],
                    out_hbm.at[cid].at[pl.ds(sid * RPW, RPW)])

  return sc_scatter_fs


@functools.lru_cache(maxsize=None)
def _make_sc_deg():
  @functools.partial(
      pl.kernel,
      out_type=jax.ShapeDtypeStruct((NC, NPAD, 16), jnp.float32),
      mesh=_MESH,
      scratch_types=[
          pltpu.VMEM((NCHUNK, CH), jnp.int32),
          pltpu.VMEM((CH, 16), jnp.float32),
          pltpu.SemaphoreType.DMA,
          pltpu.VMEM_SHARED((NPAD, 16), jnp.float32),
      ],
      compiler_params=pltpu.CompilerParams(use_tc_tiling_on_sc=False),
  )
  def sc_deg(t_hbm, ones_hbm, z_hbm, out_hbm, tv, ones_v, sem, acc):
    cid = lax.axis_index("c")
    sid = lax.axis_index("s")
    wid = sid * NC + cid
    pltpu.sync_copy(t_hbm.at[wid], tv)
    pltpu.sync_copy(ones_hbm, ones_v)
    pltpu.sync_copy(z_hbm.at[pl.ds(sid * RPW, RPW)],
                    acc.at[pl.ds(sid * RPW, RPW)])
    plsc.subcore_barrier()

    @pl.loop(0, NCHUNK)
    def _(j):
      pltpu.async_copy(ones_v, acc.at[tv.at[j]], sem, add=True)

    @pl.loop(0, NCHUNK)
    def _(j):
      pltpu.make_async_copy(z_hbm.at[pl.ds(0, CH)], ones_v, sem).wait()

    plsc.subcore_barrier()
    pltpu.sync_copy(acc.at[pl.ds(sid * RPW, RPW)],
                    out_hbm.at[cid].at[pl.ds(sid * RPW, RPW)])

  return sc_deg


_RB = 1000
_KB = 1920
_NRB = N // _RB
_NKB = 9600 // _KB


def _ffn_body(x_ref, w1_ref, b1_ref, w2_ref, b2_ref, dp_ref, wc1_ref,
              hp_ref, dinv_ref, acc_ref):
  k = pl.program_id(1)
  h = jnp.maximum(
      jnp.dot(x_ref[...], w1_ref[...], preferred_element_type=jnp.float32)
      + b1_ref[...], 0.0)
  p = jnp.dot(h.astype(jnp.bfloat16), w2_ref[...],
              preferred_element_type=jnp.float32)

  @pl.when(k == 0)
  def _():
    acc_ref[...] = p

  @pl.when(k > 0)
  def _():
    acc_ref[...] += p

  @pl.when(k == _NKB - 1)
  def _():
    o = acc_ref[...] + b2_ref[...]
    dinv = lax.rsqrt(dp_ref[0, :, 0:1] + dp_ref[1, :, 0:1] + 1.0)
    dinv_ref[...] = dinv
    hp_ref[...] = jnp.dot(
        o, wc1_ref[...], preferred_element_type=jnp.float32) * dinv


def _tc_ffn(x, w1f, b1f, w2f, b2f, degp, wc1):
  return pl.pallas_call(
      _ffn_body,
      grid=(_NRB, _NKB),
      in_specs=[
          pl.BlockSpec((_RB, 40), lambda i, k: (i, 0)),
          pl.BlockSpec((40, _KB), lambda i, k: (0, k)),
          pl.BlockSpec((1, _KB), lambda i, k: (0, k)),
          pl.BlockSpec((_KB, 40), lambda i, k: (k, 0)),
          pl.BlockSpec((1, 40), lambda i, k: (0, 0)),
          pl.BlockSpec((NC, _RB, 16), lambda i, k: (0, i, 0)),
          pl.BlockSpec((40, 64), lambda i, k: (0, 0)),
      ],
      out_specs=(
          pl.BlockSpec((_RB, 64), lambda i, k: (i, 0)),
          pl.BlockSpec((_RB, 1), lambda i, k: (i, 0)),
      ),
      out_shape=(
          jax.ShapeDtypeStruct((NPAD, 64), jnp.float32),
          jax.ShapeDtypeStruct((NPAD, 1), jnp.float32),
      ),
      scratch_shapes=[pltpu.VMEM((_RB, 40), jnp.float32)],
      compiler_params=pltpu.CompilerParams(
          dimension_semantics=("parallel", "arbitrary")),
  )(x.astype(jnp.bfloat16), w1f.astype(jnp.bfloat16), b1f,
    w2f.astype(jnp.bfloat16), b2f, degp, wc1)


def _rowmask(v):
  rows = lax.broadcasted_iota(jnp.int32, v.shape, 0)
  return jnp.where(rows < N, v, 0.0)


@functools.lru_cache(maxsize=None)
def _make_tc_mid(fin_s, fin, fout, fout_s, cat=False):
  def mid_body(ap, hp, dv, b, w, o_ref):
    if cat:
      x = jnp.concatenate([ap[0], ap[1]], 1) + hp[...]
    else:
      x = (ap[0] + ap[1] + hp[...])[:, :fin]
    dinv = dv[...]
    o = jnp.maximum(x * dinv + b[...], 0.0)
    g = jnp.dot(o, w[...], preferred_element_type=jnp.float32) * dinv
    if fout_s > fout:
      g = jnp.concatenate(
          [g, jnp.zeros((NPAD, fout_s - fout), jnp.float32)], 1)
    o_ref[...] = _rowmask(g)

  return pl.pallas_call(
      mid_body,
      out_shape=jax.ShapeDtypeStruct((NPAD, fout_s), jnp.float32),
  )


def _last_body(ap, hp, dv, b, o_ref):
  x = (ap[0] + ap[1] + hp[...])[:N, :4]
  o_ref[...] = jnp.maximum(x * dv[:N, :] + b[...], 0.0)


def _tc_last(accp, hp, dinv, b):
  return pl.pallas_call(
      _last_body,
      out_shape=jax.ShapeDtypeStruct((N, 4), jnp.float32),
  )(accp, hp, dinv, b)


def _fc_body(a_ref, w_ref, b_ref, o_ref):
  o_ref[...] = (
      jnp.dot(a_ref[...], w_ref[...], preferred_element_type=jnp.float32)
      + b_ref[...])


def _tc_fc(a, w, b):
  return pl.pallas_call(
      _fc_body,
      out_shape=jax.ShapeDtypeStruct((a.shape[0], w.shape[1]), jnp.float32),
  )(a, w, b)


def kernel(x, edge_index, batch, W1, b1, g1, be1, W2, b2, g2, be2,
           Wc1, bc1, Wc2, bc2, Wc3, bc3, Wc4, bc4, Wc5, bc5, Wfc, bfc):
  f32 = jnp.float32
  c = 1.0 / jnp.sqrt(jnp.asarray(1.0 + EPS, f32))
  w1f = W1 * (c * g1)[None, :]
  b1f = (b1 * c * g1 + be1)[None, :]
  w2f = W2 * (c * g2)[None, :]
  b2f = (b2 * c * g2 + be2)[None, :]

  s = edge_index[0].astype(jnp.int32)
  t = edge_index[1].astype(jnp.int32)
  pad = jnp.full((EP - E,), N, jnp.int32)
  s_r = jnp.concatenate([s, pad]).reshape(NW, NCHUNK, CH)
  t_r = jnp.concatenate([t, pad]).reshape(NW, NCHUNK, CH)
  s_r2 = s_r.reshape(NS, NCHUNK2, CH)
  t_r2 = t_r.reshape(NS, NCHUNK2, CH)

  ones16 = jnp.ones((CH, 16), f32)
  z16 = jnp.zeros((NPAD, 16), f32)

  degp = _make_sc_deg()(t_r, ones16, z16)
  hp, dinv = _tc_ffn(x, w1f, b1f, w2f, b2f, degp, Wc1)

  z32 = jnp.zeros((NPAD, 32), f32)
  accp = _make_sc_scatter_fsplit()(hp, s_r2, t_r2, z32)
  hp = _make_tc_mid(64, 64, 32, 32, cat=True)(
      accp, hp, dinv, bc1[None, :], Wc2)

  dims = [(32, 32, 16, 16, bc2, Wc3),
          (16, 16, 8, 16, bc3, Wc4),
          (16, 8, 4, 16, bc4, Wc5)]
  for fin_s, fin, fout, fout_s, bcl, wcl in dims:
    accp = _make_sc_scatter(fin_s)(
        hp, s_r, t_r, jnp.zeros((NPAD, fin_s), f32))
    hp = _make_tc_mid(fin_s, fin, fout, fout_s)(
        accp, hp, dinv, bcl[None, :], wcl)

  accp = _make_sc_scatter(16)(hp, s_r, t_r, z16)
  o5 = _tc_last(accp, hp, dinv, bc5[None, :])
  return _tc_fc(o5.reshape(-1, 1000 * 4), Wfc, bfc[None, :])
